# Initial kernel scaffold; baseline (speedup 1.0000x reference)
import jax
import jax.numpy as jnp
from jax.experimental import pallas as pl


def kernel(x):
    raise NotImplementedError("write your pallas kernel here")


# speedup vs baseline: 132.6024x; 132.6024x over previous
import functools

import jax
import jax.numpy as jnp
from jax import lax
from jax.experimental import pallas as pl
from jax.experimental.pallas import tpu as pltpu
from jax.experimental.pallas import tpu_sc as plsc

N = 1048576
NC = 2
NS = 16
NW = NC * NS
C = N // NW
L = 16
J = C // L

_mesh = plsc.VectorSubcoreMesh(core_axis_name="c", subcore_axis_name="s")


@functools.partial(
    pl.kernel,
    mesh=_mesh,
    out_type=[
        jax.ShapeDtypeStruct((NC, L), jnp.int32),
        jax.ShapeDtypeStruct((NC, L), jnp.int32),
    ],
    scratch_types=[
        pltpu.VMEM((C + L,), jnp.float32),
        pltpu.VMEM((L,), jnp.int32),
        pltpu.VMEM((L,), jnp.int32),
        pltpu.VMEM((NS * L,), jnp.int32),
        pltpu.VMEM((NS * L,), jnp.int32),
        pltpu.VMEM_SHARED((NS * L,), jnp.int32),
        pltpu.VMEM_SHARED((NS * L,), jnp.int32),
    ],
)
def _uc_kernel(x_hbm, flag_hbm, cnt_hbm, buf, stage_f, stage_c,
               rd_f, rd_c, sh_f, sh_c):
    c = lax.axis_index("c")
    s = lax.axis_index("s")
    w = c * NS + s
    base = w * C
    ones = jnp.full((L,), 1, jnp.int32)
    zeros = jnp.full((L,), 0, jnp.int32)

    @pl.when(w < NW - 1)
    def _():
        pltpu.sync_copy(x_hbm.at[pl.ds(base, C + 8)], buf.at[pl.ds(0, C + 8)])

    @pl.when(w == NW - 1)
    def _():
        pltpu.sync_copy(x_hbm.at[pl.ds(base, C)], buf.at[pl.ds(0, C)])
        buf[pl.ds(C, L)] = buf[pl.ds(C - 1, L)]

    v0 = buf[pl.ds(0, L)]
    first_ok = v0 == v0

    def body(j, carry):
        acc, cnt = carry
        a = buf[pl.ds(j * L, L)]
        b = buf[pl.ds(j * L + 1, L)]
        neq = a != b
        acc = acc & (b == b)
        cnt = cnt + jnp.where(neq, ones, zeros)
        return acc, cnt

    acc, cnt = lax.fori_loop(0, J, body, (first_ok, jnp.zeros((L,), jnp.int32)))

    stage_f[...] = jnp.where(acc, ones, zeros)
    stage_c[...] = cnt
    pltpu.sync_copy(stage_f, sh_f.at[pl.ds(s * L, L)])
    pltpu.sync_copy(stage_c, sh_c.at[pl.ds(s * L, L)])
    plsc.subcore_barrier()

    @pl.when(s == 0)
    def _():
        pltpu.sync_copy(sh_f, rd_f)
        pltpu.sync_copy(sh_c, rd_c)

        def red(k, carry):
            f, t = carry
            f = jnp.minimum(f, rd_f[pl.ds(k * L, L)])
            t = t + rd_c[pl.ds(k * L, L)]
            return f, t

        f, t = lax.fori_loop(0, NS, red, (ones, jnp.zeros((L,), jnp.int32)))
        f = jnp.minimum(f, jnp.where(t == t, ones, zeros))
        stage_f[...] = f
        stage_c[...] = t
        pltpu.sync_copy(stage_f, flag_hbm.at[c])
        pltpu.sync_copy(stage_c, cnt_hbm.at[c])


def kernel(x):
    flags, _counts = _uc_kernel(x)
    return jnp.all(flags != 0)


# speedup vs baseline: 146.3546x; 1.1037x over previous
import functools

import jax
import jax.numpy as jnp
from jax import lax
from jax.experimental import pallas as pl
from jax.experimental.pallas import tpu as pltpu
from jax.experimental.pallas import tpu_sc as plsc

N = 1048576
NC = 2
NS = 16
NW = NC * NS
C = N // NW
L = 16
J = C // L

_mesh = plsc.VectorSubcoreMesh(core_axis_name="c", subcore_axis_name="s")


@functools.partial(
    pl.kernel,
    mesh=_mesh,
    out_type=[
        jax.ShapeDtypeStruct((NC, L), jnp.int32),
        jax.ShapeDtypeStruct((NC, L), jnp.int32),
    ],
    scratch_types=[
        pltpu.VMEM((C + L,), jnp.float32),
        pltpu.VMEM((L,), jnp.int32),
        pltpu.VMEM((L,), jnp.int32),
        pltpu.VMEM((NS * L,), jnp.int32),
        pltpu.VMEM((NS * L,), jnp.int32),
        pltpu.VMEM_SHARED((NS * L,), jnp.int32),
        pltpu.VMEM_SHARED((NS * L,), jnp.int32),
    ],
)
def _uc_kernel(x_hbm, flag_hbm, cnt_hbm, buf, stage_f, stage_c,
               rd_f, rd_c, sh_f, sh_c):
    c = lax.axis_index("c")
    s = lax.axis_index("s")
    w = c * NS + s
    base = w * C
    ones = jnp.full((L,), 1, jnp.int32)
    zeros = jnp.full((L,), 0, jnp.int32)

    @pl.when(w < NW - 1)
    def _():
        pltpu.sync_copy(x_hbm.at[pl.ds(base, C + 8)], buf.at[pl.ds(0, C + 8)])

    @pl.when(w == NW - 1)
    def _():
        pltpu.sync_copy(x_hbm.at[pl.ds(base, C)], buf.at[pl.ds(0, C)])
        buf[pl.ds(C, L)] = buf[pl.ds(C - 1, L)]

    v0 = buf[pl.ds(0, L)]
    first_ok = v0 == v0

    U = 8

    def body(j, carry):
        acc, cnt = carry
        for k in range(U):
            a = buf[pl.ds((j * U + k) * L, L)]
            b = buf[pl.ds((j * U + k) * L + 1, L)]
            neq = a != b
            acc = acc & (b == b)
            cnt = cnt + jnp.where(neq, ones, zeros)
        return acc, cnt

    acc, cnt = lax.fori_loop(0, J // U, body,
                             (first_ok, jnp.zeros((L,), jnp.int32)))

    stage_f[...] = jnp.where(acc, ones, zeros)
    stage_c[...] = cnt
    pltpu.sync_copy(stage_f, sh_f.at[pl.ds(s * L, L)])
    pltpu.sync_copy(stage_c, sh_c.at[pl.ds(s * L, L)])
    plsc.subcore_barrier()

    @pl.when(s == 0)
    def _():
        pltpu.sync_copy(sh_f, rd_f)
        pltpu.sync_copy(sh_c, rd_c)

        def red(k, carry):
            f, t = carry
            f = jnp.minimum(f, rd_f[pl.ds(k * L, L)])
            t = t + rd_c[pl.ds(k * L, L)]
            return f, t

        f, t = lax.fori_loop(0, NS, red, (ones, jnp.zeros((L,), jnp.int32)))
        f = jnp.minimum(f, jnp.where(t == t, ones, zeros))
        stage_f[...] = f
        stage_c[...] = t
        pltpu.sync_copy(stage_f, flag_hbm.at[c])
        pltpu.sync_copy(stage_c, cnt_hbm.at[c])


def kernel(x):
    flags, _counts = _uc_kernel(x)
    return jnp.all(flags != 0)
